# Initial kernel scaffold; baseline (speedup 1.0000x reference)
import jax
import jax.numpy as jnp
from jax.experimental import pallas as pl


def kernel(edge_index, etypes, type_ids, emb_table, W_e, b_e, W_ih, W_hh, b_ih, b_hh, W_gate, b_gate, W_out, b_out):
    raise NotImplementedError("write your pallas kernel here")


# speedup vs baseline: 12.6638x; 12.6638x over previous
import functools

import jax
import jax.numpy as jnp
from jax import lax
from jax.experimental import pallas as pl
from jax.experimental.pallas import tpu as pltpu
from jax.experimental.pallas import tpu_sc as plsc

N = 10000
E = 320000
H = 128
T = 3
N_STEPS = 6

NC = 2
NS = 16
NW = NC * NS

K = 128
Q = (E + NW * K - 1) // (NW * K)
EPW = Q * K
E_PAD = EPW * NW

RPT = 632
ACC_ROWS = RPT * NS


def _edge_aggregate(table, etypes_p, src_p, dst_p, zeros_rpt):
    mesh = plsc.VectorSubcoreMesh(core_axis_name="c", subcore_axis_name="s")

    @functools.partial(
        pl.kernel,
        out_type=jax.ShapeDtypeStruct((NC, ACC_ROWS, H), jnp.float32),
        mesh=mesh,
        scratch_types=[
            pltpu.VMEM((K,), jnp.int32),
            pltpu.VMEM((K,), jnp.int32),
            pltpu.VMEM((K,), jnp.int32),
            pltpu.VMEM((K,), jnp.int32),
            pltpu.VMEM((K, H), jnp.float32),
            pltpu.VMEM_SHARED((ACC_ROWS, H), jnp.float32),
            pltpu.SemaphoreType.DMA,
        ],
    )
    def body(table_hbm, et_hbm, src_hbm, dst_hbm, z_hbm, out_hbm,
             et_v, src_v, gidx_v, dst_v, rows_v, acc_sh, sem):
        cid = lax.axis_index("c")
        sid = lax.axis_index("s")
        wid = cid * NS + sid

        pltpu.sync_copy(z_hbm, acc_sh.at[pl.ds(sid * RPT, RPT)])
        plsc.subcore_barrier()

        base = wid * EPW

        def chunk(j, _):
            off = base + j * K
            pltpu.sync_copy(et_hbm.at[pl.ds(off, K)], et_v)
            pltpu.sync_copy(src_hbm.at[pl.ds(off, K)], src_v)
            pltpu.sync_copy(dst_hbm.at[pl.ds(off, K)], dst_v)
            for i in range(K // 16):
                sl = pl.ds(i * 16, 16)
                gidx_v[sl] = et_v[sl] * N + src_v[sl]
            pltpu.async_copy(table_hbm.at[gidx_v], rows_v, sem).wait()
            pltpu.sync_copy(rows_v, acc_sh.at[dst_v], add=True)
            return 0

        lax.fori_loop(0, Q, chunk, 0)
        plsc.subcore_barrier()
        pltpu.sync_copy(acc_sh.at[pl.ds(sid * RPT, RPT)],
                        out_hbm.at[cid, pl.ds(sid * RPT, RPT)])

    return body(table, etypes_p, src_p, dst_p, zeros_rpt)


def _tc_embed(type_ids, emb_table):
    def body(ids_ref, emb_ref, o_ref):
        ids = ids_ref[...]
        onehot = (ids[:, None] == lax.broadcasted_iota(jnp.int32, (N, 128), 1)
                  ).astype(jnp.float32)
        o_ref[...] = jnp.dot(onehot, emb_ref[...],
                             preferred_element_type=jnp.float32)

    emb_pad = jnp.zeros((128, H), jnp.float32).at[:100].set(emb_table)
    return pl.pallas_call(
        body,
        out_shape=jax.ShapeDtypeStruct((N, H), jnp.float32),
    )(type_ids, emb_pad)


def _tc_transform(h, W_e, b_e):
    BN = 1000

    def body(h_ref, w_ref, b_ref, o_ref):
        o_ref[0] = (jnp.dot(h_ref[...], w_ref[0],
                            preferred_element_type=jnp.float32)
                    + b_ref[0])

    return pl.pallas_call(
        body,
        grid=(T, N // BN),
        in_specs=[
            pl.BlockSpec((BN, H), lambda t, i: (i, 0)),
            pl.BlockSpec((1, H, H), lambda t, i: (t, 0, 0)),
            pl.BlockSpec((1, 1, H), lambda t, i: (t, 0, 0)),
        ],
        out_specs=pl.BlockSpec((1, BN, H), lambda t, i: (t, i, 0)),
        out_shape=jax.ShapeDtypeStruct((T, N, H), jnp.float32),
    )(h, W_e, b_e.reshape(T, 1, H))


def _tc_gru(p0, p1, h, W_ihT, W_hhT, b_ih, b_hh):
    BN = 1000

    def body(p0_ref, p1_ref, h_ref, wi_ref, wh_ref, bi_ref, bh_ref, o_ref):
        a = p0_ref[...] + p1_ref[...]
        hh = h_ref[...]
        gi = jnp.dot(a, wi_ref[...], preferred_element_type=jnp.float32) + bi_ref[...]
        gh = jnp.dot(hh, wh_ref[...], preferred_element_type=jnp.float32) + bh_ref[...]
        r = jax.nn.sigmoid(gi[:, :H] + gh[:, :H])
        z = jax.nn.sigmoid(gi[:, H:2 * H] + gh[:, H:2 * H])
        n = jnp.tanh(gi[:, 2 * H:] + r * gh[:, 2 * H:])
        o_ref[...] = (1.0 - z) * n + z * hh

    return pl.pallas_call(
        body,
        grid=(N // BN,),
        in_specs=[
            pl.BlockSpec((BN, H), lambda i: (i, 0)),
            pl.BlockSpec((BN, H), lambda i: (i, 0)),
            pl.BlockSpec((BN, H), lambda i: (i, 0)),
            pl.BlockSpec((H, 3 * H), lambda i: (0, 0)),
            pl.BlockSpec((H, 3 * H), lambda i: (0, 0)),
            pl.BlockSpec((1, 3 * H), lambda i: (0, 0)),
            pl.BlockSpec((1, 3 * H), lambda i: (0, 0)),
        ],
        out_specs=pl.BlockSpec((BN, H), lambda i: (i, 0)),
        out_shape=jax.ShapeDtypeStruct((N, H), jnp.float32),
    )(p0, p1, h, W_ihT, W_hhT, b_ih, b_hh)


def _tc_pool(h, ann, wg1, wg2, b_gate, wo1, wo2, b_out):
    OUT = b_out.shape[-1]

    def body(h_ref, a_ref, wg1_ref, wg2_ref, bg_ref, wo1_ref, wo2_ref, bo_ref,
             o_ref):
        hh = h_ref[...]
        aa = a_ref[...]
        lg = (jnp.dot(hh, wg1_ref[...], preferred_element_type=jnp.float32)
              + jnp.dot(aa, wg2_ref[...], preferred_element_type=jnp.float32)
              + bg_ref[0, 0])
        m = jnp.max(lg)
        e = jnp.exp(lg - m)
        g = e / jnp.sum(e)
        rh = jnp.sum(g * hh, axis=0, keepdims=True)
        ra = jnp.sum(g * aa, axis=0, keepdims=True)
        o_ref[...] = (jnp.dot(rh, wo1_ref[...], preferred_element_type=jnp.float32)
                      + jnp.dot(ra, wo2_ref[...], preferred_element_type=jnp.float32)
                      + bo_ref[...])

    return pl.pallas_call(
        body,
        out_shape=jax.ShapeDtypeStruct((1, OUT), jnp.float32),
    )(h, ann, wg1, wg2, b_gate, wo1, wo2, b_out)


def kernel(edge_index, etypes, type_ids, emb_table, W_e, b_e, W_ih, W_hh,
           b_ih, b_hh, W_gate, b_gate, W_out, b_out):
    src = edge_index[0]
    dst = edge_index[1]
    pad = E_PAD - E
    et_p = jnp.concatenate([etypes, jnp.zeros((pad,), jnp.int32)])
    src_p = jnp.concatenate([src, jnp.zeros((pad,), jnp.int32)])
    dst_p = jnp.concatenate([dst, jnp.full((pad,), ACC_ROWS - 1, jnp.int32)])
    zeros_rpt = jnp.zeros((RPT, H), jnp.float32)

    W_ihT = W_ih.T
    W_hhT = W_hh.T
    b_ih2 = b_ih.reshape(1, 3 * H)
    b_hh2 = b_hh.reshape(1, 3 * H)
    wg1 = W_gate[:H]
    wg2 = W_gate[H:]
    wo1 = W_out[:H]
    wo2 = W_out[H:]
    bg2 = b_gate.reshape(1, 1)
    bo2 = b_out.reshape(1, -1)

    ann = _tc_embed(type_ids, emb_table)
    h = ann
    for _ in range(N_STEPS):
        all_t = _tc_transform(h, W_e, b_e)
        table = all_t.reshape(T * N, H)
        parts = _edge_aggregate(table, et_p, src_p, dst_p, zeros_rpt)
        p0 = parts[0, :N]
        p1 = parts[1, :N]
        h = _tc_gru(p0, p1, h, W_ihT, W_hhT, b_ih2, b_hh2)
    return _tc_pool(h, ann, wg1, wg2, bg2, wo1, wo2, bo2)
